# baseline (speedup 1.0000x reference)
import dataclasses
import functools

import jax
import jax.numpy as jnp
from jax import lax
from jax.experimental import pallas as pl
from jax.experimental.pallas import tpu as pltpu
from jax.experimental.pallas import tpu_sc as plsc

K = 5
B = 128
N = 100000
BLK = 12544
NB = (N + BLK - 1) // BLK
W = 16
ROWS_PER_TILE = 16
NTILES = B // ROWS_PER_TILE

_sc_gather_cache = []


def _sc_gather_body(ypred_hbm, y_hbm, out_hbm, y_v, win_v, sem):
    wid = lax.axis_index("s") * 2 + lax.axis_index("c")

    @pl.when(wid < NTILES)
    def _():
        base = wid * ROWS_PER_TILE
        pltpu.sync_copy(y_hbm.at[pl.ds(base, ROWS_PER_TILE)], y_v)
        yvec = y_v[...]
        rid = lax.iota(jnp.int32, 16)
        for r in range(ROWS_PER_TILE):
            er = jnp.sum(jnp.where(rid == r, yvec, 0))
            off = (er // W) * W
            pltpu.async_copy(
                ypred_hbm.at[base + r, pl.ds(off, W)], win_v.at[r], sem
            )
        for r in range(ROWS_PER_TILE):
            pltpu.make_async_copy(
                ypred_hbm.at[0, pl.ds(0, W)], win_v.at[r], sem
            ).wait()
        pltpu.sync_copy(win_v, out_hbm.at[pl.ds(base, ROWS_PER_TILE), :])


def _sc_gather(y_pred, y32):
    if not _sc_gather_cache:
        mesh = plsc.VectorSubcoreMesh(
            core_axis_name="c", subcore_axis_name="s", num_cores=2, num_subcores=16
        )
        cp = pltpu.CompilerParams()
        if "needs_layout_passes" in pltpu.CompilerParams.__dataclass_fields__:
            cp = dataclasses.replace(cp, needs_layout_passes=False)
        _sc_gather_cache.append(
            pl.kernel(
                _sc_gather_body,
                out_type=jax.ShapeDtypeStruct((B, W), jnp.float32),
                mesh=mesh,
                compiler_params=cp,
                scratch_types=[
                    pltpu.VMEM((ROWS_PER_TILE,), jnp.int32),
                    pltpu.VMEM((ROWS_PER_TILE, W), jnp.float32),
                    pltpu.SemaphoreType.DMA,
                ],
            )
        )
    return _sc_gather_cache[0](y_pred, y32)


def _tc_body(y_vec_ref, g_ref, x_ref, out_ref, v_ref, cnt_ref):
    j = pl.program_id(0)
    yv = y_vec_ref[...]
    x = x_ref[...]
    col = j * BLK + jax.lax.broadcasted_iota(jnp.int32, (B, BLK), 1)

    @pl.when(j == 0)
    def _init():
        lane = yv - (yv // W) * W
        lw = jax.lax.broadcasted_iota(jnp.int32, (B, W), 1)
        v_ref[...] = jnp.sum(
            jnp.where(lw == lane, g_ref[...], 0.0), axis=1, keepdims=True
        )
        cnt_ref[...] = jnp.zeros((B, 1), jnp.float32)

    v = v_ref[...]

    @pl.when(j < NB - 1)
    def _count():
        hits = jnp.logical_or(
            x > v, jnp.logical_and(x == v, col < yv)
        ).astype(jnp.float32)
        cnt_ref[...] += jnp.sum(hits, axis=1, keepdims=True)

    @pl.when(j == NB - 1)
    def _tail():
        xm = jnp.where(col < N, x, -jnp.inf)
        hits = jnp.logical_or(
            xm > v, jnp.logical_and(xm == v, col < yv)
        ).astype(jnp.float32)
        cnt_ref[...] += jnp.sum(hits, axis=1, keepdims=True)
        in_topk = (cnt_ref[...] < float(K)).astype(jnp.int32)
        out_ref[0, 0] = jnp.sum(in_topk)


def _tc_count(y_vec, g, y_pred):
    out = pl.pallas_call(
        _tc_body,
        grid=(NB,),
        in_specs=[
            pl.BlockSpec((B, 1), lambda j: (0, 0)),
            pl.BlockSpec((B, W), lambda j: (0, 0)),
            pl.BlockSpec((B, BLK), lambda j: (0, j)),
        ],
        out_specs=pl.BlockSpec(memory_space=pltpu.MemorySpace.SMEM),
        out_shape=jax.ShapeDtypeStruct((1, 1), jnp.int32),
        scratch_shapes=[
            pltpu.VMEM((B, 1), jnp.float32),
            pltpu.VMEM((B, 1), jnp.float32),
        ],
        compiler_params=pltpu.CompilerParams(
            dimension_semantics=("arbitrary",),
        ),
    )(y_vec, g, y_pred)
    return out.reshape(())


def kernel(y_pred, y):
    y32 = y.astype(jnp.int32)
    g = _sc_gather(y_pred, y32)
    return _tc_count(y32.reshape(B, 1), g, y_pred)


# speedup vs baseline: 1.0257x; 1.0257x over previous
import dataclasses
import functools

import jax
import jax.numpy as jnp
from jax import lax
from jax.experimental import pallas as pl
from jax.experimental.pallas import tpu as pltpu
from jax.experimental.pallas import tpu_sc as plsc

K = 5
B = 128
N = 100000
RB = 16
NRB = B // RB
W = 16
ROWS_PER_TILE = 16
NTILES = B // ROWS_PER_TILE

_sc_gather_cache = []


def _sc_gather_body(ypred_hbm, y_hbm, out_hbm, y_v, win_v, sem):
    wid = lax.axis_index("s") * 2 + lax.axis_index("c")

    @pl.when(wid < NTILES)
    def _():
        base = wid * ROWS_PER_TILE
        pltpu.sync_copy(y_hbm.at[pl.ds(base, ROWS_PER_TILE)], y_v)
        yvec = y_v[...]
        rid = lax.iota(jnp.int32, 16)
        for r in range(ROWS_PER_TILE):
            er = jnp.sum(jnp.where(rid == r, yvec, 0))
            off = (er // W) * W
            pltpu.async_copy(
                ypred_hbm.at[base + r, pl.ds(off, W)], win_v.at[r], sem
            )
        for r in range(ROWS_PER_TILE):
            pltpu.make_async_copy(
                ypred_hbm.at[0, pl.ds(0, W)], win_v.at[r], sem
            ).wait()
        pltpu.sync_copy(win_v, out_hbm.at[pl.ds(base, ROWS_PER_TILE), :])


def _sc_gather(y_pred, y32):
    if not _sc_gather_cache:
        mesh = plsc.VectorSubcoreMesh(
            core_axis_name="c", subcore_axis_name="s", num_cores=2, num_subcores=16
        )
        cp = pltpu.CompilerParams()
        if "needs_layout_passes" in pltpu.CompilerParams.__dataclass_fields__:
            cp = dataclasses.replace(cp, needs_layout_passes=False)
        _sc_gather_cache.append(
            pl.kernel(
                _sc_gather_body,
                out_type=jax.ShapeDtypeStruct((B, W), jnp.float32),
                mesh=mesh,
                compiler_params=cp,
                scratch_types=[
                    pltpu.VMEM((ROWS_PER_TILE,), jnp.int32),
                    pltpu.VMEM((ROWS_PER_TILE, W), jnp.float32),
                    pltpu.SemaphoreType.DMA,
                ],
            )
        )
    return _sc_gather_cache[0](y_pred, y32)


def _tc_body(y_vec_ref, g_ref, x_ref, out_ref):
    j = pl.program_id(0)
    yv = y_vec_ref[...]
    x = x_ref[...]

    lane = yv - (yv // W) * W
    lw = jax.lax.broadcasted_iota(jnp.int32, (RB, W), 1)
    v = jnp.sum(jnp.where(lw == lane, g_ref[...], 0.0), axis=1, keepdims=True)

    io = jax.lax.broadcasted_iota(jnp.int32, (RB, N), 1)
    hits = jnp.logical_or(
        x > v, jnp.logical_and(x == v, io < yv)
    ).astype(jnp.float32)
    cnt = jnp.sum(hits, axis=1, keepdims=True)
    part = jnp.sum((cnt < float(K)).astype(jnp.int32))

    @pl.when(j == 0)
    def _first():
        out_ref[0, 0] = part

    @pl.when(j > 0)
    def _rest():
        out_ref[0, 0] += part


def _tc_count(y_vec, g, y_pred):
    out = pl.pallas_call(
        _tc_body,
        grid=(NRB,),
        in_specs=[
            pl.BlockSpec((RB, 1), lambda j: (j, 0)),
            pl.BlockSpec((RB, W), lambda j: (j, 0)),
            pl.BlockSpec((RB, N), lambda j: (j, 0)),
        ],
        out_specs=pl.BlockSpec(memory_space=pltpu.MemorySpace.SMEM),
        out_shape=jax.ShapeDtypeStruct((1, 1), jnp.int32),
        compiler_params=pltpu.CompilerParams(
            dimension_semantics=("arbitrary",),
        ),
    )(y_vec, g, y_pred)
    return out.reshape(())


def kernel(y_pred, y):
    y32 = y.astype(jnp.int32)
    g = _sc_gather(y_pred, y32)
    return _tc_count(y32.reshape(B, 1), g, y_pred)


# speedup vs baseline: 1.1961x; 1.1661x over previous
import jax
import jax.numpy as jnp
from jax.experimental import pallas as pl
from jax.experimental.pallas import tpu as pltpu

K = 5
B = 128
N = 100000
RB = 32
NRB = B // RB


def _tc_body(y_vec_ref, x_ref, out_ref):
    j = pl.program_id(0)
    yv = y_vec_ref[...]
    x = x_ref[...]
    io = jax.lax.broadcasted_iota(jnp.int32, (RB, N), 1)

    eqy = io == yv
    v = jnp.max(jnp.where(eqy, x, -jnp.inf), axis=1, keepdims=True)
    hits = jnp.logical_or(
        x > v, jnp.logical_and(x == v, io < yv)
    ).astype(jnp.float32)
    cnt = jnp.sum(hits, axis=1, keepdims=True)
    part = jnp.sum((cnt < float(K)).astype(jnp.int32))

    @pl.when(j == 0)
    def _first():
        out_ref[0, 0] = part

    @pl.when(j > 0)
    def _rest():
        out_ref[0, 0] += part


def kernel(y_pred, y):
    y_vec = y.astype(jnp.int32).reshape(B, 1)
    out = pl.pallas_call(
        _tc_body,
        grid=(NRB,),
        in_specs=[
            pl.BlockSpec((RB, 1), lambda j: (j, 0)),
            pl.BlockSpec((RB, N), lambda j: (j, 0)),
        ],
        out_specs=pl.BlockSpec(memory_space=pltpu.MemorySpace.SMEM),
        out_shape=jax.ShapeDtypeStruct((1, 1), jnp.int32),
        compiler_params=pltpu.CompilerParams(
            dimension_semantics=("arbitrary",),
        ),
    )(y_vec, y_pred)
    return out.reshape(())
